# baseline (speedup 1.0000x reference)
import functools

import jax
import jax.numpy as jnp
from jax import lax
from jax.experimental import pallas as pl
from jax.experimental.pallas import tpu as pltpu
from jax.experimental.pallas import tpu_sc as plsc

CW = 64
NCH = 4
ROW = 2 * CW
BATCH = 128
NSUB = 16
NWORK = 32


def _pre_node_body(node_ref, wt_ref, wq_ref, lw_ref, elw_ref,
                   src_ref, dstq_ref, c_ref, la_ref, lb_ref):
    d = node_ref.shape[1]
    nb = node_ref[...]
    wq = wq_ref[...]
    a = jnp.dot(nb, wt_ref[0:d, :], preferred_element_type=jnp.float32)
    c = jnp.dot(nb, wt_ref[2 * d:3 * d, :], preferred_element_type=jnp.float32)
    aq = jnp.dot(a, wq, preferred_element_type=jnp.float32)
    cq = jnp.dot(c, wq, preferred_element_type=jnp.float32)
    c_ref[...] = c
    la_ref[...] = jnp.dot(nb, lw_ref[...], preferred_element_type=jnp.float32)
    lb_ref[...] = jnp.dot(nb, elw_ref[...], preferred_element_type=jnp.float32)
    for ch in range(NCH):
        sl = slice(ch * CW, (ch + 1) * CW)
        src_ref[ch] = jnp.concatenate([a[:, sl], aq[:, sl]], axis=1)
        dstq_ref[ch] = jnp.concatenate([c[:, sl], cq[:, sl]], axis=1)


def _pre_rel_body(rel_ref, wt_ref, wq_ref, rel_out_ref):
    d = rel_ref.shape[1]
    rb = rel_ref[...]
    b = jnp.dot(rb, wt_ref[d:2 * d, :], preferred_element_type=jnp.float32)
    bq = jnp.dot(b, wq_ref[...], preferred_element_type=jnp.float32)
    for ch in range(NCH):
        sl = slice(ch * CW, (ch + 1) * CW)
        rel_out_ref[ch] = jnp.concatenate([b[:, sl], bq[:, sl]], axis=1)


def _combine_body(p_ref, c_ref, la_ref, lb_ref, norm_ref, out_ref):
    p = p_ref[...]
    denom = jnp.concatenate([p[ch, :, 0:CW] for ch in range(NCH)], axis=1)
    sab = jnp.concatenate([p[ch, :, CW:ROW] for ch in range(NCH)], axis=1)
    c = c_ref[...]
    s = sab + c * denom
    h = s / (denom + 1e-16) * norm_ref[...]
    loop = jnp.where(denom[:, 0:1] > 0.0, la_ref[...], lb_ref[...])
    out_ref[...] = h + loop


def _make_sc_kernel(n, e, r, nhalf):
    nbatch = e // BATCH
    nb_per = nbatch // NSUB
    accrows = nhalf + 128
    stripe = accrows // NSUB
    mesh = plsc.VectorSubcoreMesh(core_axis_name="c", subcore_axis_name="s")

    @functools.partial(
        pl.kernel,
        out_type=jax.ShapeDtypeStruct((2, NCH, accrows, ROW), jnp.float32),
        mesh=mesh,
        scratch_types=[
            pltpu.VMEM((BATCH,), jnp.int32),
            pltpu.VMEM((BATCH,), jnp.int32),
            pltpu.VMEM((BATCH,), jnp.int32),
            pltpu.VMEM((BATCH,), jnp.int32),
            pltpu.VMEM((BATCH, ROW), jnp.float32),
            pltpu.VMEM((BATCH, ROW), jnp.float32),
            pltpu.VMEM((BATCH, ROW), jnp.float32),
            pltpu.VMEM((BATCH, ROW), jnp.float32),
            pltpu.VMEM_SHARED((nhalf + 128, ROW), jnp.float32),
            pltpu.SemaphoreType.DMA,
            pltpu.SemaphoreType.DMA,
            pltpu.SemaphoreType.DMA,
        ],
    )
    def sc_edge(src_tab, dstq_tab, rel_tab, src4, dstq4, et4, dsth, zeros,
                out, sidx, dqidx, eidx, didx, srcbuf, dstbuf, relbuf,
                compbuf, acc, sem0, sem1, sem2):
        cid = lax.axis_index("c")
        sid = lax.axis_index("s")

        for ch in range(NCH):
            pltpu.sync_copy(zeros, acc.at[pl.ds(sid * stripe, stripe)])
            plsc.subcore_barrier()

            @pl.loop(0, nb_per)
            def batch_a(i):
                base = (sid + i * NSUB) * BATCH
                pltpu.sync_copy(src4.at[pl.ds(ch * e + base, BATCH)], sidx)
                pltpu.sync_copy(dstq4.at[pl.ds(ch * e + base, BATCH)], dqidx)
                pltpu.sync_copy(et4.at[pl.ds(ch * e + base, BATCH)], eidx)
                pltpu.sync_copy(dsth.at[cid, pl.ds(base, BATCH)], didx)
                cp0 = pltpu.async_copy(src_tab.at[sidx], srcbuf, sem0)
                cp1 = pltpu.async_copy(dstq_tab.at[dqidx], dstbuf, sem1)
                cp2 = pltpu.async_copy(rel_tab.at[eidx], relbuf, sem2)
                cp0.wait()
                cp1.wait()
                cp2.wait()

                @pl.loop(0, BATCH, unroll=4)
                def edge_body(j):
                    for g in range(CW // 16):
                        lo = pl.ds(g * 16, 16)
                        hi = pl.ds(CW + g * 16, 16)
                        aq = srcbuf[j, hi] + relbuf[j, hi] + dstbuf[j, hi]
                        ex = jnp.exp(jnp.maximum(aq, aq * 0.01))
                        t = srcbuf[j, lo] + relbuf[j, lo]
                        compbuf[j, lo] = ex
                        compbuf[j, hi] = ex * t

                pltpu.sync_copy(compbuf, acc.at[didx], add=True)

            plsc.subcore_barrier()
            pltpu.sync_copy(acc.at[pl.ds(sid * stripe, stripe)],
                            out.at[cid, ch, pl.ds(sid * stripe, stripe)])
            plsc.subcore_barrier()

    return sc_edge


def kernel(node, rel, edge_index, edge_type, norm, loop_weight,
           evolve_loop_weight, w_triplet, w_quad):
    n, d = node.shape
    r = rel.shape[0]
    e = edge_type.shape[0]
    src = edge_index[0]
    dst = edge_index[1]

    npad = ((n + 255) // 256) * 256
    nhalf = npad // 2
    step = NSUB * BATCH
    epad = ((e + step - 1) // step) * step
    pad = epad - e
    zpad = jnp.zeros((pad,), jnp.int32)
    srcp = jnp.concatenate([src, zpad])
    dstg = jnp.concatenate([dst, zpad])
    dstp = jnp.concatenate([dst, jnp.full((pad,), 2 * nhalf, jnp.int32)])
    etp = jnp.concatenate([edge_type, zpad])

    offs_n = (jnp.arange(NCH, dtype=jnp.int32) * n)[:, None]
    offs_r = (jnp.arange(NCH, dtype=jnp.int32) * r)[:, None]
    src4 = (srcp[None, :] + offs_n).reshape(-1)
    dstq4 = (dstg[None, :] + offs_n).reshape(-1)
    et4 = (etp[None, :] + offs_r).reshape(-1)
    dsth = jnp.stack([
        jnp.where(dstp < nhalf, dstp, nhalf),
        jnp.where((dstp >= nhalf) & (dstp < 2 * nhalf), dstp - nhalf, nhalf),
    ])

    bn = n // 10
    pre = pl.pallas_call(
        _pre_node_body,
        grid=(n // bn,),
        in_specs=[
            pl.BlockSpec((bn, d), lambda i: (i, 0)),
            pl.BlockSpec((3 * d, d), lambda i: (0, 0)),
            pl.BlockSpec((d, d), lambda i: (0, 0)),
            pl.BlockSpec((d, d), lambda i: (0, 0)),
            pl.BlockSpec((d, d), lambda i: (0, 0)),
        ],
        out_specs=[
            pl.BlockSpec((NCH, bn, ROW), lambda i: (0, i, 0)),
            pl.BlockSpec((NCH, bn, ROW), lambda i: (0, i, 0)),
            pl.BlockSpec((bn, d), lambda i: (i, 0)),
            pl.BlockSpec((bn, d), lambda i: (i, 0)),
            pl.BlockSpec((bn, d), lambda i: (i, 0)),
        ],
        out_shape=[
            jax.ShapeDtypeStruct((NCH, n, ROW), jnp.float32),
            jax.ShapeDtypeStruct((NCH, n, ROW), jnp.float32),
            jax.ShapeDtypeStruct((n, d), jnp.float32),
            jax.ShapeDtypeStruct((n, d), jnp.float32),
            jax.ShapeDtypeStruct((n, d), jnp.float32),
        ],
    )
    src_tab, dstq_tab, c_tab, la_tab, lb_tab = pre(
        node, w_triplet, w_quad, loop_weight, evolve_loop_weight)

    rel_pre = pl.pallas_call(
        _pre_rel_body,
        out_shape=jax.ShapeDtypeStruct((NCH, r, ROW), jnp.float32),
    )
    rel_tab = rel_pre(rel, w_triplet, w_quad)

    sc_edge = _make_sc_kernel(n, epad, r, nhalf)
    accrows = nhalf + 128
    zeros = jnp.zeros((accrows // NSUB, ROW), jnp.float32)
    partial = sc_edge(
        src_tab.reshape(NCH * n, ROW),
        dstq_tab.reshape(NCH * n, ROW),
        rel_tab.reshape(NCH * r, ROW),
        src4, dstq4, et4, dsth, zeros)

    parts = jnp.concatenate(
        [partial[0, :, :nhalf], partial[1, :, :n - nhalf]], axis=1)

    combine = pl.pallas_call(
        _combine_body,
        grid=(n // bn,),
        in_specs=[
            pl.BlockSpec((NCH, bn, ROW), lambda i: (0, i, 0)),
            pl.BlockSpec((bn, d), lambda i: (i, 0)),
            pl.BlockSpec((bn, d), lambda i: (i, 0)),
            pl.BlockSpec((bn, d), lambda i: (i, 0)),
            pl.BlockSpec((bn, 1), lambda i: (i, 0)),
        ],
        out_specs=pl.BlockSpec((bn, d), lambda i: (i, 0)),
        out_shape=jax.ShapeDtypeStruct((n, d), jnp.float32),
    )
    return combine(parts, c_tab, la_tab, lb_tab, norm)


# speedup vs baseline: 1.7300x; 1.7300x over previous
import functools

import jax
import jax.numpy as jnp
from jax import lax
from jax.experimental import pallas as pl
from jax.experimental.pallas import tpu as pltpu
from jax.experimental.pallas import tpu_sc as plsc

CW = 64
NCH = 4
ROW = 2 * CW
BATCH = 128
NSUB = 16
NWORK = 32


def _pre_node_body(node_ref, wt_ref, wq_ref, lw_ref, elw_ref,
                   src_ref, dstq_ref, c_ref, la_ref, lb_ref):
    d = node_ref.shape[1]
    nb = node_ref[...]
    wq = wq_ref[...]
    a = jnp.dot(nb, wt_ref[0:d, :], preferred_element_type=jnp.float32)
    c = jnp.dot(nb, wt_ref[2 * d:3 * d, :], preferred_element_type=jnp.float32)
    aq = jnp.dot(a, wq, preferred_element_type=jnp.float32)
    cq = jnp.dot(c, wq, preferred_element_type=jnp.float32)
    c_ref[...] = c
    la_ref[...] = jnp.dot(nb, lw_ref[...], preferred_element_type=jnp.float32)
    lb_ref[...] = jnp.dot(nb, elw_ref[...], preferred_element_type=jnp.float32)
    for ch in range(NCH):
        sl = slice(ch * CW, (ch + 1) * CW)
        src_ref[ch] = jnp.concatenate([a[:, sl], aq[:, sl]], axis=1)
        dstq_ref[ch] = jnp.concatenate([c[:, sl], cq[:, sl]], axis=1)


def _pre_rel_body(rel_ref, wt_ref, wq_ref, rel_out_ref):
    d = rel_ref.shape[1]
    rb = rel_ref[...]
    b = jnp.dot(rb, wt_ref[d:2 * d, :], preferred_element_type=jnp.float32)
    bq = jnp.dot(b, wq_ref[...], preferred_element_type=jnp.float32)
    for ch in range(NCH):
        sl = slice(ch * CW, (ch + 1) * CW)
        rel_out_ref[ch] = jnp.concatenate([b[:, sl], bq[:, sl]], axis=1)


def _combine_body(p_ref, c_ref, la_ref, lb_ref, norm_ref, out_ref):
    p = p_ref[...]
    denom = jnp.concatenate([p[ch, :, 0:CW] for ch in range(NCH)], axis=1)
    sab = jnp.concatenate([p[ch, :, CW:ROW] for ch in range(NCH)], axis=1)
    c = c_ref[...]
    s = sab + c * denom
    h = s / (denom + 1e-16) * norm_ref[...]
    loop = jnp.where(denom[:, 0:1] > 0.0, la_ref[...], lb_ref[...])
    out_ref[...] = h + loop


def _make_sc_kernel(n, e, r, nhalf):
    nbatch = e // BATCH
    nb_per = nbatch // NSUB
    accrows = nhalf + 128
    stripe = accrows // NSUB
    mesh = plsc.VectorSubcoreMesh(core_axis_name="c", subcore_axis_name="s")

    @functools.partial(
        pl.kernel,
        out_type=jax.ShapeDtypeStruct((2, NCH, accrows, ROW), jnp.float32),
        mesh=mesh,
        scratch_types=[
            pltpu.VMEM((BATCH,), jnp.int32),
            pltpu.VMEM((BATCH,), jnp.int32),
            pltpu.VMEM((BATCH,), jnp.int32),
            pltpu.VMEM((BATCH,), jnp.int32),
            pltpu.VMEM((BATCH, ROW), jnp.float32),
            pltpu.VMEM((BATCH, ROW), jnp.float32),
            pltpu.VMEM((BATCH, ROW), jnp.float32),
            pltpu.VMEM((BATCH, ROW), jnp.float32),
            pltpu.VMEM_SHARED((nhalf + 128, ROW), jnp.float32),
            pltpu.SemaphoreType.DMA,
            pltpu.SemaphoreType.DMA,
            pltpu.SemaphoreType.DMA,
        ],
    )
    def sc_edge(src_tab, dstq_tab, rel_tab, src4, dstq4, et4, dsth, zeros,
                out, sidx, dqidx, eidx, didx, srcbuf, dstbuf, relbuf,
                compbuf, acc, sem0, sem1, sem2):
        cid = lax.axis_index("c")
        sid = lax.axis_index("s")

        for ch in range(NCH):
            pltpu.sync_copy(zeros, acc.at[pl.ds(sid * stripe, stripe)])
            plsc.subcore_barrier()

            @pl.loop(0, nb_per)
            def batch_a(i):
                base = (sid + i * NSUB) * BATCH
                pltpu.sync_copy(src4.at[pl.ds(ch * e + base, BATCH)], sidx)
                pltpu.sync_copy(dstq4.at[pl.ds(ch * e + base, BATCH)], dqidx)
                pltpu.sync_copy(et4.at[pl.ds(ch * e + base, BATCH)], eidx)
                pltpu.sync_copy(dsth.at[cid, pl.ds(base, BATCH)], didx)
                cp0 = pltpu.async_copy(src_tab.at[sidx], srcbuf, sem0)
                cp1 = pltpu.async_copy(dstq_tab.at[dqidx], dstbuf, sem1)
                cp2 = pltpu.async_copy(rel_tab.at[eidx], relbuf, sem2)
                cp0.wait()
                cp1.wait()
                cp2.wait()

                @pl.loop(0, BATCH)
                def edge_body(j):
                    for g in range(CW // 16):
                        lo = pl.ds(g * 16, 16)
                        hi = pl.ds(CW + g * 16, 16)
                        aq = srcbuf[j, hi] + relbuf[j, hi] + dstbuf[j, hi]
                        ex = jnp.exp(jnp.maximum(aq, aq * 0.01))
                        t = srcbuf[j, lo] + relbuf[j, lo]
                        compbuf[j, lo] = ex
                        compbuf[j, hi] = ex * t

                pltpu.sync_copy(compbuf, acc.at[didx], add=True)

            plsc.subcore_barrier()
            pltpu.sync_copy(acc.at[pl.ds(sid * stripe, stripe)],
                            out.at[cid, ch, pl.ds(sid * stripe, stripe)])
            plsc.subcore_barrier()

    return sc_edge


def kernel(node, rel, edge_index, edge_type, norm, loop_weight,
           evolve_loop_weight, w_triplet, w_quad):
    n, d = node.shape
    r = rel.shape[0]
    e = edge_type.shape[0]
    src = edge_index[0]
    dst = edge_index[1]

    npad = ((n + 255) // 256) * 256
    nhalf = npad // 2
    step = NSUB * BATCH
    epad = ((e + step - 1) // step) * step
    pad = epad - e
    zpad = jnp.zeros((pad,), jnp.int32)
    srcp = jnp.concatenate([src, zpad])
    dstg = jnp.concatenate([dst, zpad])
    dstp = jnp.concatenate([dst, jnp.full((pad,), 2 * nhalf, jnp.int32)])
    etp = jnp.concatenate([edge_type, zpad])

    offs_n = (jnp.arange(NCH, dtype=jnp.int32) * n)[:, None]
    offs_r = (jnp.arange(NCH, dtype=jnp.int32) * r)[:, None]
    src4 = (srcp[None, :] + offs_n).reshape(-1)
    dstq4 = (dstg[None, :] + offs_n).reshape(-1)
    et4 = (etp[None, :] + offs_r).reshape(-1)
    dsth = jnp.stack([
        jnp.where(dstp < nhalf, dstp, nhalf),
        jnp.where((dstp >= nhalf) & (dstp < 2 * nhalf), dstp - nhalf, nhalf),
    ])

    bn = n // 10
    pre = pl.pallas_call(
        _pre_node_body,
        grid=(n // bn,),
        in_specs=[
            pl.BlockSpec((bn, d), lambda i: (i, 0)),
            pl.BlockSpec((3 * d, d), lambda i: (0, 0)),
            pl.BlockSpec((d, d), lambda i: (0, 0)),
            pl.BlockSpec((d, d), lambda i: (0, 0)),
            pl.BlockSpec((d, d), lambda i: (0, 0)),
        ],
        out_specs=[
            pl.BlockSpec((NCH, bn, ROW), lambda i: (0, i, 0)),
            pl.BlockSpec((NCH, bn, ROW), lambda i: (0, i, 0)),
            pl.BlockSpec((bn, d), lambda i: (i, 0)),
            pl.BlockSpec((bn, d), lambda i: (i, 0)),
            pl.BlockSpec((bn, d), lambda i: (i, 0)),
        ],
        out_shape=[
            jax.ShapeDtypeStruct((NCH, n, ROW), jnp.float32),
            jax.ShapeDtypeStruct((NCH, n, ROW), jnp.float32),
            jax.ShapeDtypeStruct((n, d), jnp.float32),
            jax.ShapeDtypeStruct((n, d), jnp.float32),
            jax.ShapeDtypeStruct((n, d), jnp.float32),
        ],
    )
    src_tab, dstq_tab, c_tab, la_tab, lb_tab = pre(
        node, w_triplet, w_quad, loop_weight, evolve_loop_weight)

    rel_pre = pl.pallas_call(
        _pre_rel_body,
        out_shape=jax.ShapeDtypeStruct((NCH, r, ROW), jnp.float32),
    )
    rel_tab = rel_pre(rel, w_triplet, w_quad)

    sc_edge = _make_sc_kernel(n, epad, r, nhalf)
    accrows = nhalf + 128
    zeros = jnp.zeros((accrows // NSUB, ROW), jnp.float32)
    partial = sc_edge(
        src_tab.reshape(NCH * n, ROW),
        dstq_tab.reshape(NCH * n, ROW),
        rel_tab.reshape(NCH * r, ROW),
        src4, dstq4, et4, dsth, zeros)

    parts = jnp.concatenate(
        [partial[0, :, :nhalf], partial[1, :, :n - nhalf]], axis=1)

    combine = pl.pallas_call(
        _combine_body,
        grid=(n // bn,),
        in_specs=[
            pl.BlockSpec((NCH, bn, ROW), lambda i: (0, i, 0)),
            pl.BlockSpec((bn, d), lambda i: (i, 0)),
            pl.BlockSpec((bn, d), lambda i: (i, 0)),
            pl.BlockSpec((bn, d), lambda i: (i, 0)),
            pl.BlockSpec((bn, 1), lambda i: (i, 0)),
        ],
        out_specs=pl.BlockSpec((bn, d), lambda i: (i, 0)),
        out_shape=jax.ShapeDtypeStruct((n, d), jnp.float32),
    )
    return combine(parts, c_tab, la_tab, lb_tab, norm)
